# baseline (speedup 1.0000x reference)
import jax
import jax.numpy as jnp
from jax import lax
from jax.experimental import pallas as pl
from jax.experimental.pallas import tpu as pltpu
from jax.experimental.pallas import tpu_sc as plsc
from jax._src.pallas import mpmd as _mpmd

_FEATURE = 1024
_QSIZE = 256
_NCLASS = 100
_BATCH = 4096

_NC = 2
_NS = 16
_L = 16
_NW = _NC * _NS
_CPAD = 128
_CHUNK = 64


def _norm_body(emb_ref, out_ref):
    x = emb_ref[...]
    n = jnp.sqrt(jnp.sum(x * x, axis=1, keepdims=True))
    out_ref[...] = x / jnp.maximum(n, 1e-12)


def _normalize(embeddings):
    blk = 512
    return pl.pallas_call(
        _norm_body,
        grid=(_BATCH // blk,),
        in_specs=[pl.BlockSpec((blk, _FEATURE), lambda i: (i, 0))],
        out_specs=pl.BlockSpec((blk, _FEATURE), lambda i: (i, 0)),
        out_shape=jax.ShapeDtypeStruct((_BATCH, _FEATURE), jnp.float32),
    )(embeddings)


def _copy_body(src_ref, dst_ref):
    dst_ref[...] = src_ref[...]


def _copy(queue2d):
    blk = 1024
    nrows = _NCLASS * _QSIZE
    return pl.pallas_call(
        _copy_body,
        grid=(nrows // blk,),
        in_specs=[pl.BlockSpec((blk, _FEATURE), lambda i: (i, 0))],
        out_specs=pl.BlockSpec((blk, _FEATURE), lambda i: (i, 0)),
        out_shape=jax.ShapeDtypeStruct((nrows, _FEATURE), jnp.float32),
    )(queue2d)


def _sc_body(labels_hbm, ptr_hbm, qcopy_hbm, emb_hbm,
             out_hbm, newptr_hbm,
             labels_v, ptr_v, match_v, rows_v, ptrbuf_v,
             srcidx_v, dstidx_v, row_sem):
    del qcopy_hbm
    wid = lax.axis_index("s") * _NC + lax.axis_index("c")
    pltpu.sync_copy(labels_hbm, labels_v)
    pltpu.sync_copy(ptr_hbm, ptr_v)
    lane = lax.iota(jnp.int32, _L)

    for k in range(_CPAD // _NW):
        c = wid + _NW * k

        @pl.when(c < _NCLASS)
        def _process(c=c):
            def scan_step(i, cnt):
                lbl = labels_v[pl.ds(i * _L, _L)]
                msk = lbl == c
                inc = plsc.cumsum(msk.astype(jnp.int32))
                plsc.store_scatter(match_v, [cnt + inc - 1], lane + i * _L,
                                   mask=msk)
                return cnt + jnp.max(inc)

            cnt = lax.fori_loop(0, _BATCH // _L, scan_step, 0)

            pv = ptr_v[pl.ds((c // _L) * _L, _L)]
            ptr_c = jnp.sum(jnp.where(lane == lax.rem(c, _L), pv, 0))

            newp = lax.rem(ptr_c + cnt, _QSIZE)
            ptrbuf_v[...] = jnp.broadcast_to(newp, (_L,))
            pltpu.sync_copy(ptrbuf_v, newptr_hbm.at[c])

            def chunk_step(j, _):
                for t in range(_CHUNK // _L):
                    r = lane + j * _CHUNK + t * _L
                    rc = jnp.minimum(r, cnt - 1)
                    src = plsc.load_gather(match_v, [rc])
                    dst = c * _QSIZE + lax.rem(ptr_c + rc, _QSIZE)
                    srcidx_v[pl.ds(t * _L, _L)] = src
                    dstidx_v[pl.ds(t * _L, _L)] = dst
                pltpu.async_copy(emb_hbm.at[srcidx_v], rows_v, row_sem).wait()
                pltpu.async_copy(rows_v, out_hbm.at[dstidx_v], row_sem).wait()
                return 0

            nchunks = lax.div(cnt + _CHUNK - 1, _CHUNK)
            lax.fori_loop(0, nchunks, chunk_step, 0)


_sc_update = _mpmd._mpmd_map(
    [(plsc.VectorSubcoreMesh(core_axis_name="c", subcore_axis_name="s"),
      _sc_body)],
    (
        jax.ShapeDtypeStruct((_NCLASS * _QSIZE, _FEATURE), jnp.float32),
        jax.ShapeDtypeStruct((_CPAD, _L), jnp.int32),
    ),
    input_output_aliases={2: 0},
    scratch_types=[
        pltpu.VMEM((_BATCH,), jnp.int32),
        pltpu.VMEM((_CPAD,), jnp.int32),
        pltpu.VMEM((_QSIZE + _L,), jnp.int32),
        pltpu.VMEM((_CHUNK, _FEATURE), jnp.float32),
        pltpu.VMEM((_L,), jnp.int32),
        pltpu.VMEM((_CHUNK,), jnp.int32),
        pltpu.VMEM((_CHUNK,), jnp.int32),
        pltpu.SemaphoreType.DMA,
    ],
    compiler_params=pltpu.CompilerParams(needs_layout_passes=False),
)


def kernel(embeddings, labels, queue, queue_ptr):
    emb_norm = _normalize(embeddings.astype(jnp.float32))
    ptr_pad = jnp.pad(queue_ptr, (0, _CPAD - _NCLASS))
    queue2d = queue.reshape(_NCLASS * _QSIZE, _FEATURE)
    qcopy = _copy(queue2d)
    out2d, newptr_pad = _sc_update(labels, ptr_pad, qcopy, emb_norm)
    return (out2d.reshape(_NCLASS, _QSIZE, _FEATURE), newptr_pad[:_NCLASS, 0])


# speedup vs baseline: 1.0676x; 1.0676x over previous
import jax
import jax.numpy as jnp
from jax import lax
from jax.experimental import pallas as pl
from jax.experimental.pallas import tpu as pltpu
from jax.experimental.pallas import tpu_sc as plsc
from jax._src.pallas import mpmd as _mpmd

_FEATURE = 1024
_QSIZE = 256
_NCLASS = 100
_BATCH = 4096

_NC = 2
_NS = 16
_L = 16
_NW = _NC * _NS
_CPAD = 128
_CHUNK = 64


def _norm_body(emb_ref, out_ref):
    x = emb_ref[...]
    n = jnp.sqrt(jnp.sum(x * x, axis=1, keepdims=True))
    out_ref[...] = x / jnp.maximum(n, 1e-12)


def _normalize(embeddings):
    blk = 512
    return pl.pallas_call(
        _norm_body,
        grid=(_BATCH // blk,),
        in_specs=[pl.BlockSpec((blk, _FEATURE), lambda i: (i, 0))],
        out_specs=pl.BlockSpec((blk, _FEATURE), lambda i: (i, 0)),
        out_shape=jax.ShapeDtypeStruct((_BATCH, _FEATURE), jnp.float32),
    )(embeddings)


def _copy_body(src_ref, dst_ref):
    dst_ref[...] = src_ref[...]


def _copy(queue2d):
    blk = 1024
    nrows = _NCLASS * _QSIZE
    return pl.pallas_call(
        _copy_body,
        grid=(nrows // blk,),
        in_specs=[pl.BlockSpec((blk, _FEATURE), lambda i: (i, 0))],
        out_specs=pl.BlockSpec((blk, _FEATURE), lambda i: (i, 0)),
        out_shape=jax.ShapeDtypeStruct((nrows, _FEATURE), jnp.float32),
    )(queue2d)


def _sc_body(labels_hbm, ptr_hbm, qcopy_hbm, emb_hbm,
             out_hbm, newptr_hbm,
             labels_v, ptr_v, match_v, rows_v, ptrbuf_v, row_sem):
    del qcopy_hbm
    wid = lax.axis_index("s") * _NC + lax.axis_index("c")
    pltpu.sync_copy(labels_hbm, labels_v)
    pltpu.sync_copy(ptr_hbm, ptr_v)
    lane = lax.iota(jnp.int32, _L)

    for k in range(_CPAD // _NW):
        c = wid + _NW * k

        @pl.when(c < _NCLASS)
        def _process(c=c):
            def scan_step(i, cnt):
                lbl = labels_v[pl.ds(i * _L, _L)]
                msk = lbl == c
                inc = plsc.cumsum(msk.astype(jnp.int32))
                plsc.store_scatter(match_v, [cnt + inc - 1], lane + i * _L,
                                   mask=msk)
                return cnt + jnp.max(inc)

            cnt = lax.fori_loop(0, _BATCH // _L, scan_step, 0)

            pv = ptr_v[pl.ds((c // _L) * _L, _L)]
            ptr_c = jnp.sum(jnp.where(lane == lax.rem(c, _L), pv, 0))

            newp = lax.rem(ptr_c + cnt, _QSIZE)
            ptrbuf_v[...] = jnp.broadcast_to(newp, (_L,))
            pltpu.sync_copy(ptrbuf_v, newptr_hbm.at[c])

            def chunk_step(j, _):
                r = lane + j * _L
                rc = jnp.minimum(r, cnt - 1)
                src = plsc.load_gather(match_v, [rc])
                dst = c * _QSIZE + lax.rem(ptr_c + rc, _QSIZE)
                pltpu.async_copy(emb_hbm.at[src], rows_v, row_sem).wait()
                pltpu.async_copy(rows_v, out_hbm.at[dst], row_sem).wait()
                return 0

            nchunks = lax.div(cnt + _L - 1, _L)
            lax.fori_loop(0, nchunks, chunk_step, 0)


_sc_update = _mpmd._mpmd_map(
    [(plsc.VectorSubcoreMesh(core_axis_name="c", subcore_axis_name="s"),
      _sc_body)],
    (
        jax.ShapeDtypeStruct((_NCLASS * _QSIZE, _FEATURE), jnp.float32),
        jax.ShapeDtypeStruct((_CPAD, _L), jnp.int32),
    ),
    input_output_aliases={2: 0},
    scratch_types=[
        pltpu.VMEM((_BATCH,), jnp.int32),
        pltpu.VMEM((_CPAD,), jnp.int32),
        pltpu.VMEM((_QSIZE + _L,), jnp.int32),
        pltpu.VMEM((_L, _FEATURE), jnp.float32),
        pltpu.VMEM((_L,), jnp.int32),
        pltpu.SemaphoreType.DMA,
    ],
    compiler_params=pltpu.CompilerParams(needs_layout_passes=False),
)


def kernel(embeddings, labels, queue, queue_ptr):
    emb_norm = _normalize(embeddings.astype(jnp.float32))
    ptr_pad = jnp.pad(queue_ptr, (0, _CPAD - _NCLASS))
    queue2d = queue.reshape(_NCLASS * _QSIZE, _FEATURE)
    qcopy = _copy(queue2d)
    out2d, newptr_pad = _sc_update(labels, ptr_pad, qcopy, emb_norm)
    return (out2d.reshape(_NCLASS, _QSIZE, _FEATURE), newptr_pad[:_NCLASS, 0])


# speedup vs baseline: 1.2054x; 1.1291x over previous
import jax
import jax.numpy as jnp
from jax import lax
from jax.experimental import pallas as pl
from jax.experimental.pallas import tpu as pltpu
from jax.experimental.pallas import tpu_sc as plsc
from jax._src.pallas import mpmd as _mpmd

_FEATURE = 1024
_QSIZE = 256
_NCLASS = 100
_BATCH = 4096

_NC = 2
_NS = 16
_L = 16
_NW = _NC * _NS
_CPAD = 128
_CHUNK = 64


def _norm_body(emb_ref, out_ref):
    x = emb_ref[...]
    n = jnp.sqrt(jnp.sum(x * x, axis=1, keepdims=True))
    out_ref[...] = x / jnp.maximum(n, 1e-12)


def _normalize(embeddings):
    blk = 512
    return pl.pallas_call(
        _norm_body,
        grid=(_BATCH // blk,),
        in_specs=[pl.BlockSpec((blk, _FEATURE), lambda i: (i, 0))],
        out_specs=pl.BlockSpec((blk, _FEATURE), lambda i: (i, 0)),
        out_shape=jax.ShapeDtypeStruct((_BATCH, _FEATURE), jnp.float32),
    )(embeddings)


def _copy_body(src_ref, dst_ref):
    dst_ref[...] = src_ref[...]


def _copy(queue2d):
    blk = 1024
    nrows = _NCLASS * _QSIZE
    return pl.pallas_call(
        _copy_body,
        grid=(nrows // blk,),
        in_specs=[pl.BlockSpec((blk, _FEATURE), lambda i: (i, 0))],
        out_specs=pl.BlockSpec((blk, _FEATURE), lambda i: (i, 0)),
        out_shape=jax.ShapeDtypeStruct((nrows, _FEATURE), jnp.float32),
    )(queue2d)


def _sc_body(labels_hbm, ptr_hbm, qcopy_hbm, emb_hbm,
             out_hbm, newptr_hbm,
             labels_v, ptr_v, cmb_v, lblcmb_v, match_v, rows_v, ptrbuf_v,
             gsem, ssem):
    del qcopy_hbm
    wid = lax.axis_index("s") * _NC + lax.axis_index("c")
    pltpu.sync_copy(labels_hbm, labels_v)
    pltpu.sync_copy(ptr_hbm, ptr_v)
    lane = lax.iota(jnp.int32, _L)

    def scan1(i, mcnt):
        lbl = labels_v[pl.ds(i * _L, _L)]
        msk = jnp.bitwise_and(lbl, _NW - 1) == wid
        inc = plsc.cumsum(msk.astype(jnp.int32))
        pos = mcnt + inc - 1
        plsc.store_scatter(cmb_v, [pos], lane + i * _L, mask=msk)
        plsc.store_scatter(lblcmb_v, [pos], lbl, mask=msk)
        return mcnt + jnp.max(inc)

    mcnt = lax.fori_loop(0, _BATCH // _L, scan1, 0)
    mvecs = lax.div(mcnt + _L - 1, _L)

    for k in range(_CPAD // _NW):
        c = wid + _NW * k

        @pl.when(c < _NCLASS)
        def _process(c=c):
            def scan2(i, cnt):
                l2 = lblcmb_v[pl.ds(i * _L, _L)]
                b2 = cmb_v[pl.ds(i * _L, _L)]
                msk = (l2 == c) & (lane + i * _L < mcnt)
                inc = plsc.cumsum(msk.astype(jnp.int32))
                plsc.store_scatter(match_v, [cnt + inc - 1], b2, mask=msk)
                return cnt + jnp.max(inc)

            cnt = lax.fori_loop(0, mvecs, scan2, 0)

            pv = ptr_v[pl.ds((c // _L) * _L, _L)]
            ptr_c = jnp.sum(jnp.where(lane == lax.rem(c, _L), pv, 0))

            newp = lax.rem(ptr_c + cnt, _QSIZE)
            ptrbuf_v[...] = jnp.broadcast_to(newp, (_L,))
            pltpu.sync_copy(ptrbuf_v, newptr_hbm.at[c])

            nchunks = lax.div(cnt + _L - 1, _L)

            def fire_gather(j):
                r = lane + j * _L
                rc = jnp.minimum(r, cnt - 1)
                src = plsc.load_gather(match_v, [rc])
                buf = lax.rem(j, 2)
                pltpu.make_async_copy(emb_hbm.at[src], rows_v.at[buf],
                                      gsem).start()

            @pl.when(nchunks > 0)
            def _prime():
                fire_gather(0)

            def chunk_step(j, _):
                @pl.when(j > 0)
                def _drain_prev_scatter():
                    pltpu.make_async_copy(
                        rows_v.at[lax.rem(j - 1, 2)],
                        out_hbm.at[pl.ds(0, _L)], ssem).wait()

                @pl.when(j + 1 < nchunks)
                def _next_gather():
                    fire_gather(j + 1)

                buf = lax.rem(j, 2)
                pltpu.make_async_copy(
                    emb_hbm.at[pl.ds(0, _L)], rows_v.at[buf], gsem).wait()
                r = lane + j * _L
                rc = jnp.minimum(r, cnt - 1)
                dst = c * _QSIZE + lax.rem(ptr_c + rc, _QSIZE)
                pltpu.make_async_copy(rows_v.at[buf], out_hbm.at[dst],
                                      ssem).start()
                return 0

            lax.fori_loop(0, nchunks, chunk_step, 0)

            @pl.when(nchunks > 0)
            def _drain_last():
                pltpu.make_async_copy(
                    rows_v.at[lax.rem(nchunks - 1, 2)],
                    out_hbm.at[pl.ds(0, _L)], ssem).wait()


_sc_update = _mpmd._mpmd_map(
    [(plsc.VectorSubcoreMesh(core_axis_name="c", subcore_axis_name="s"),
      _sc_body)],
    (
        jax.ShapeDtypeStruct((_NCLASS * _QSIZE, _FEATURE), jnp.float32),
        jax.ShapeDtypeStruct((_CPAD, _L), jnp.int32),
    ),
    input_output_aliases={2: 0},
    scratch_types=[
        pltpu.VMEM((_BATCH,), jnp.int32),
        pltpu.VMEM((_CPAD,), jnp.int32),
        pltpu.VMEM((_BATCH + _L,), jnp.int32),
        pltpu.VMEM((_BATCH + _L,), jnp.int32),
        pltpu.VMEM((_QSIZE + _L,), jnp.int32),
        pltpu.VMEM((2, _L, _FEATURE), jnp.float32),
        pltpu.VMEM((_L,), jnp.int32),
        pltpu.SemaphoreType.DMA,
        pltpu.SemaphoreType.DMA,
    ],
    compiler_params=pltpu.CompilerParams(needs_layout_passes=False),
)


def kernel(embeddings, labels, queue, queue_ptr):
    emb_norm = _normalize(embeddings.astype(jnp.float32))
    ptr_pad = jnp.pad(queue_ptr, (0, _CPAD - _NCLASS))
    queue2d = queue.reshape(_NCLASS * _QSIZE, _FEATURE)
    qcopy = _copy(queue2d)
    out2d, newptr_pad = _sc_update(labels, ptr_pad, qcopy, emb_norm)
    return (out2d.reshape(_NCLASS, _QSIZE, _FEATURE), newptr_pad[:_NCLASS, 0])


# speedup vs baseline: 1.2314x; 1.0216x over previous
import jax
import jax.numpy as jnp
from jax import lax
from jax.experimental import pallas as pl
from jax.experimental.pallas import tpu as pltpu
from jax.experimental.pallas import tpu_sc as plsc
from jax._src.pallas import mpmd as _mpmd

_FEATURE = 1024
_QSIZE = 256
_NCLASS = 100
_BATCH = 4096

_NC = 2
_NS = 16
_L = 16
_NW = _NC * _NS
_CPAD = 128
_CHUNK = 64


_QBLK = 1024
_EBLK = 512
_QSTEPS = _NCLASS * _QSIZE // _QBLK
_ESTEPS = _BATCH // _EBLK


def _tc_body(q_ref, emb_ref, qout_ref, eout_ref):
    i = pl.program_id(0)
    qout_ref[...] = q_ref[...]

    @pl.when(i < _ESTEPS)
    def _norm():
        x = emb_ref[...]
        n = jnp.sqrt(jnp.sum(x * x, axis=1, keepdims=True))
        eout_ref[...] = x / jnp.maximum(n, 1e-12)


def _tc_fused(queue2d, embeddings):
    nrows = _NCLASS * _QSIZE
    return pl.pallas_call(
        _tc_body,
        grid=(_QSTEPS,),
        in_specs=[
            pl.BlockSpec((_QBLK, _FEATURE), lambda i: (i, 0)),
            pl.BlockSpec((_EBLK, _FEATURE),
                         lambda i: (jnp.minimum(i, _ESTEPS - 1), 0)),
        ],
        out_specs=[
            pl.BlockSpec((_QBLK, _FEATURE), lambda i: (i, 0)),
            pl.BlockSpec((_EBLK, _FEATURE),
                         lambda i: (jnp.minimum(i, _ESTEPS - 1), 0)),
        ],
        out_shape=[
            jax.ShapeDtypeStruct((nrows, _FEATURE), jnp.float32),
            jax.ShapeDtypeStruct((_BATCH, _FEATURE), jnp.float32),
        ],
    )(queue2d, embeddings)


def _sc_body(labels_hbm, ptr_hbm, qcopy_hbm, emb_hbm,
             out_hbm, newptr_hbm,
             labels_v, ptr_v, cmb_v, lblcmb_v, match_v, rows_v, ptrbuf_v,
             gsem, ssem):
    del qcopy_hbm
    wid = lax.axis_index("s") * _NC + lax.axis_index("c")
    pltpu.sync_copy(labels_hbm, labels_v)
    pltpu.sync_copy(ptr_hbm, ptr_v)
    lane = lax.iota(jnp.int32, _L)

    def scan1(i, mcnt):
        lbl = labels_v[pl.ds(i * _L, _L)]
        msk = jnp.bitwise_and(lbl, _NW - 1) == wid
        inc = plsc.cumsum(msk.astype(jnp.int32))
        pos = mcnt + inc - 1
        plsc.store_scatter(cmb_v, [pos], lane + i * _L, mask=msk)
        plsc.store_scatter(lblcmb_v, [pos], lbl, mask=msk)
        return mcnt + jnp.max(inc)

    mcnt = lax.fori_loop(0, _BATCH // _L, scan1, 0)
    mvecs = lax.div(mcnt + _L - 1, _L)

    for k in range(_CPAD // _NW):
        c = wid + _NW * k

        @pl.when(c < _NCLASS)
        def _process(c=c):
            def scan2(i, cnt):
                l2 = lblcmb_v[pl.ds(i * _L, _L)]
                b2 = cmb_v[pl.ds(i * _L, _L)]
                msk = (l2 == c) & (lane + i * _L < mcnt)
                inc = plsc.cumsum(msk.astype(jnp.int32))
                plsc.store_scatter(match_v, [cnt + inc - 1], b2, mask=msk)
                return cnt + jnp.max(inc)

            cnt = lax.fori_loop(0, mvecs, scan2, 0)

            pv = ptr_v[pl.ds((c // _L) * _L, _L)]
            ptr_c = jnp.sum(jnp.where(lane == lax.rem(c, _L), pv, 0))

            newp = lax.rem(ptr_c + cnt, _QSIZE)
            ptrbuf_v[...] = jnp.broadcast_to(newp, (_L,))
            pltpu.sync_copy(ptrbuf_v, newptr_hbm.at[c])

            nchunks = lax.div(cnt + _L - 1, _L)

            def fire_gather(j):
                r = lane + j * _L
                rc = jnp.minimum(r, cnt - 1)
                src = plsc.load_gather(match_v, [rc])
                buf = lax.rem(j, 2)
                pltpu.make_async_copy(emb_hbm.at[src], rows_v.at[buf],
                                      gsem).start()

            @pl.when(nchunks > 0)
            def _prime():
                fire_gather(0)

            def chunk_step(j, _):
                @pl.when(j > 0)
                def _drain_prev_scatter():
                    pltpu.make_async_copy(
                        rows_v.at[lax.rem(j - 1, 2)],
                        out_hbm.at[pl.ds(0, _L)], ssem).wait()

                @pl.when(j + 1 < nchunks)
                def _next_gather():
                    fire_gather(j + 1)

                buf = lax.rem(j, 2)
                pltpu.make_async_copy(
                    emb_hbm.at[pl.ds(0, _L)], rows_v.at[buf], gsem).wait()
                r = lane + j * _L
                rc = jnp.minimum(r, cnt - 1)
                dst = c * _QSIZE + lax.rem(ptr_c + rc, _QSIZE)
                pltpu.make_async_copy(rows_v.at[buf], out_hbm.at[dst],
                                      ssem).start()
                return 0

            lax.fori_loop(0, nchunks, chunk_step, 0)

            @pl.when(nchunks > 0)
            def _drain_last():
                pltpu.make_async_copy(
                    rows_v.at[lax.rem(nchunks - 1, 2)],
                    out_hbm.at[pl.ds(0, _L)], ssem).wait()


_sc_update = _mpmd._mpmd_map(
    [(plsc.VectorSubcoreMesh(core_axis_name="c", subcore_axis_name="s"),
      _sc_body)],
    (
        jax.ShapeDtypeStruct((_NCLASS * _QSIZE, _FEATURE), jnp.float32),
        jax.ShapeDtypeStruct((_CPAD, _L), jnp.int32),
    ),
    input_output_aliases={2: 0},
    scratch_types=[
        pltpu.VMEM((_BATCH,), jnp.int32),
        pltpu.VMEM((_CPAD,), jnp.int32),
        pltpu.VMEM((_BATCH + _L,), jnp.int32),
        pltpu.VMEM((_BATCH + _L,), jnp.int32),
        pltpu.VMEM((_QSIZE + _L,), jnp.int32),
        pltpu.VMEM((2, _L, _FEATURE), jnp.float32),
        pltpu.VMEM((_L,), jnp.int32),
        pltpu.SemaphoreType.DMA,
        pltpu.SemaphoreType.DMA,
    ],
    compiler_params=pltpu.CompilerParams(needs_layout_passes=False),
)


def kernel(embeddings, labels, queue, queue_ptr):
    ptr_pad = jnp.pad(queue_ptr, (0, _CPAD - _NCLASS))
    queue2d = queue.reshape(_NCLASS * _QSIZE, _FEATURE)
    qcopy, emb_norm = _tc_fused(queue2d, embeddings)
    out2d, newptr_pad = _sc_update(labels, ptr_pad, qcopy, emb_norm)
    return (out2d.reshape(_NCLASS, _QSIZE, _FEATURE), newptr_pad[:_NCLASS, 0])


# speedup vs baseline: 1.2393x; 1.0064x over previous
import jax
import jax.numpy as jnp
from jax import lax
from jax.experimental import pallas as pl
from jax.experimental.pallas import tpu as pltpu
from jax.experimental.pallas import tpu_sc as plsc
from jax._src.pallas import mpmd as _mpmd

_FEATURE = 1024
_QSIZE = 256
_NCLASS = 100
_BATCH = 4096

_NC = 2
_NS = 16
_L = 16
_NW = _NC * _NS
_CPAD = 128
_CHUNK = 64


_QBLK = 1600
_EBLK = 256
_QSTEPS = _NCLASS * _QSIZE // _QBLK
_ESTEPS = _BATCH // _EBLK


def _tc_body(q_ref, emb_ref, qout_ref, eout_ref):
    i = pl.program_id(0)
    qout_ref[...] = q_ref[...]

    @pl.when(i < _ESTEPS)
    def _norm():
        x = emb_ref[...]
        n = jnp.sqrt(jnp.sum(x * x, axis=1, keepdims=True))
        eout_ref[...] = x / jnp.maximum(n, 1e-12)


def _tc_fused(queue2d, embeddings):
    nrows = _NCLASS * _QSIZE
    return pl.pallas_call(
        _tc_body,
        grid=(_QSTEPS,),
        in_specs=[
            pl.BlockSpec((_QBLK, _FEATURE), lambda i: (i, 0)),
            pl.BlockSpec((_EBLK, _FEATURE),
                         lambda i: (jnp.minimum(i, _ESTEPS - 1), 0)),
        ],
        out_specs=[
            pl.BlockSpec((_QBLK, _FEATURE), lambda i: (i, 0)),
            pl.BlockSpec((_EBLK, _FEATURE),
                         lambda i: (jnp.minimum(i, _ESTEPS - 1), 0)),
        ],
        out_shape=[
            jax.ShapeDtypeStruct((nrows, _FEATURE), jnp.float32),
            jax.ShapeDtypeStruct((_BATCH, _FEATURE), jnp.float32),
        ],
    )(queue2d, embeddings)


def _sc_body(labels_hbm, ptr_hbm, qcopy_hbm, emb_hbm,
             out_hbm, newptr_hbm,
             labels_v, ptr_v, cmb_v, lblcmb_v, match_v, rows_v, ptrbuf_v,
             gsem, ssem):
    del qcopy_hbm
    wid = lax.axis_index("s") * _NC + lax.axis_index("c")
    pltpu.sync_copy(labels_hbm, labels_v)
    pltpu.sync_copy(ptr_hbm, ptr_v)
    lane = lax.iota(jnp.int32, _L)

    def scan1(i, mcnt):
        lbl = labels_v[pl.ds(i * _L, _L)]
        msk = jnp.bitwise_and(lbl, _NW - 1) == wid
        inc = plsc.cumsum(msk.astype(jnp.int32))
        pos = mcnt + inc - 1
        plsc.store_scatter(cmb_v, [pos], lane + i * _L, mask=msk)
        plsc.store_scatter(lblcmb_v, [pos], lbl, mask=msk)
        return mcnt + jnp.max(inc)

    mcnt = lax.fori_loop(0, _BATCH // _L, scan1, 0)
    mvecs = lax.div(mcnt + _L - 1, _L)

    for k in range(_CPAD // _NW):
        c = wid + _NW * k

        @pl.when(c < _NCLASS)
        def _process(c=c):
            def scan2(i, cnt):
                l2 = lblcmb_v[pl.ds(i * _L, _L)]
                b2 = cmb_v[pl.ds(i * _L, _L)]
                msk = (l2 == c) & (lane + i * _L < mcnt)
                inc = plsc.cumsum(msk.astype(jnp.int32))
                plsc.store_scatter(match_v, [cnt + inc - 1], b2, mask=msk)
                return cnt + jnp.max(inc)

            cnt = lax.fori_loop(0, mvecs, scan2, 0)

            pv = ptr_v[pl.ds((c // _L) * _L, _L)]
            ptr_c = jnp.sum(jnp.where(lane == lax.rem(c, _L), pv, 0))

            newp = lax.rem(ptr_c + cnt, _QSIZE)
            ptrbuf_v[...] = jnp.broadcast_to(newp, (_L,))
            pltpu.sync_copy(ptrbuf_v, newptr_hbm.at[c])

            nchunks = lax.div(cnt + _L - 1, _L)

            def fire_gather(j):
                r = lane + j * _L
                rc = jnp.minimum(r, cnt - 1)
                src = plsc.load_gather(match_v, [rc])
                buf = lax.rem(j, 2)
                pltpu.make_async_copy(emb_hbm.at[src], rows_v.at[buf],
                                      gsem).start()

            @pl.when(nchunks > 0)
            def _prime():
                fire_gather(0)

            def chunk_step(j, _):
                @pl.when(j > 0)
                def _drain_prev_scatter():
                    pltpu.make_async_copy(
                        rows_v.at[lax.rem(j - 1, 2)],
                        out_hbm.at[pl.ds(0, _L)], ssem).wait()

                @pl.when(j + 1 < nchunks)
                def _next_gather():
                    fire_gather(j + 1)

                buf = lax.rem(j, 2)
                pltpu.make_async_copy(
                    emb_hbm.at[pl.ds(0, _L)], rows_v.at[buf], gsem).wait()
                r = lane + j * _L
                rc = jnp.minimum(r, cnt - 1)
                dst = c * _QSIZE + lax.rem(ptr_c + rc, _QSIZE)
                pltpu.make_async_copy(rows_v.at[buf], out_hbm.at[dst],
                                      ssem).start()
                return 0

            lax.fori_loop(0, nchunks, chunk_step, 0)

            @pl.when(nchunks > 0)
            def _drain_last():
                pltpu.make_async_copy(
                    rows_v.at[lax.rem(nchunks - 1, 2)],
                    out_hbm.at[pl.ds(0, _L)], ssem).wait()


_sc_update = _mpmd._mpmd_map(
    [(plsc.VectorSubcoreMesh(core_axis_name="c", subcore_axis_name="s"),
      _sc_body)],
    (
        jax.ShapeDtypeStruct((_NCLASS * _QSIZE, _FEATURE), jnp.float32),
        jax.ShapeDtypeStruct((_CPAD, _L), jnp.int32),
    ),
    input_output_aliases={2: 0},
    scratch_types=[
        pltpu.VMEM((_BATCH,), jnp.int32),
        pltpu.VMEM((_CPAD,), jnp.int32),
        pltpu.VMEM((_BATCH + _L,), jnp.int32),
        pltpu.VMEM((_BATCH + _L,), jnp.int32),
        pltpu.VMEM((_QSIZE + _L,), jnp.int32),
        pltpu.VMEM((2, _L, _FEATURE), jnp.float32),
        pltpu.VMEM((_L,), jnp.int32),
        pltpu.SemaphoreType.DMA,
        pltpu.SemaphoreType.DMA,
    ],
    compiler_params=pltpu.CompilerParams(needs_layout_passes=False),
)


def kernel(embeddings, labels, queue, queue_ptr):
    ptr_pad = jnp.pad(queue_ptr, (0, _CPAD - _NCLASS))
    queue2d = queue.reshape(_NCLASS * _QSIZE, _FEATURE)
    qcopy, emb_norm = _tc_fused(queue2d, embeddings)
    out2d, newptr_pad = _sc_update(labels, ptr_pad, qcopy, emb_norm)
    return (out2d.reshape(_NCLASS, _QSIZE, _FEATURE), newptr_pad[:_NCLASS, 0])
